# baseline (speedup 1.0000x reference)
import jax
import jax.numpy as jnp
from jax.experimental import pallas as pl
from jax.experimental.pallas import tpu as pltpu

B = 8
S = 2048
T = 65
E = 2048
ROWS_OUT = S + T
RB = 960
NBLK = (ROWS_OUT + RB - 1) // RB
WIN = 72


def _fused_kernel(ts_ref, mask_ref, na_ref, npos_ref, tpos_ref,
                  w1_ref, b1_ref, w2_ref, b2_ref, wt_ref,
                  wt1_ref, bt1_ref, wt2_hbm, bt2_ref,
                  in_ref, out_ref, vl_sm, dit_s, wt2_s, sem):
    b = pl.program_id(0)
    i = pl.program_id(1)

    @pl.when((b == 0) & (i == 0))
    def _():
        cp = pltpu.make_async_copy(wt2_hbm, wt2_s, sem)
        cp.start()
        for bb in range(B):
            vl_sm[bb] = jnp.sum(mask_ref[bb])
        g = na_ref[...] @ w1_ref[...] + b1_ref[...]
        g = jax.nn.gelu(g, approximate=True)
        act = g @ w2_ref[...] + b2_ref[...]
        x = ts_ref[...] * wt_ref[...]
        x = jnp.concatenate([jnp.cos(x), jnp.sin(x)], axis=1)
        h1 = x @ wt1_ref[...] + bt1_ref[...]
        h1 = h1 * jax.nn.sigmoid(h1)
        cp.wait()
        tt = h1 @ wt2_s[...] + bt2_ref[...] + tpos_ref[...]
        for bb in range(B):
            dit_s[bb] = jnp.concatenate(
                [tt[bb:bb + 1], act[64 * bb:64 * (bb + 1)] + npos_ref[...]],
                axis=0)

    vl = vl_sm[b]
    r0 = i * RB

    @pl.when(r0 + RB <= S)
    def _():
        out_ref[...] = in_ref[...]

    @pl.when(r0 + RB > S)
    def _():
        rows = r0 + jax.lax.broadcasted_iota(jnp.int32, (RB, 1), 0)
        out_ref[...] = jnp.where(rows < S, in_ref[...], 0.0)

    intersects = (vl < r0 + RB) & (vl + T > r0)

    @pl.when(intersects)
    def _():
        lo = jax.lax.max(vl, r0)
        w0 = jax.lax.min(((lo - r0) // 8) * 8, RB - WIN)
        rows = r0 + w0 + jax.lax.broadcasted_iota(jnp.int32, (WIN, 1), 0)
        rel = rows - vl
        in_window = (rel >= 0) & (rel < T)
        keep = jnp.logical_not(in_window) & (rows < S)
        sub = jnp.where(keep, in_ref[pl.ds(w0, WIN), :], 0.0)
        j = jax.lax.broadcasted_iota(jnp.int32, (WIN, T), 1)
        p = ((rel == j) & in_window).astype(jnp.float32)
        win = jax.lax.dot(p, dit_s[b],
                          preferred_element_type=jnp.float32)
        out_ref[pl.ds(w0, WIN), :] = sub + win


def _fused(noisy_actions, timesteps, attention_mask, inputs_embeds,
           noise_pos, timestep_pos, W1, b1, W2, b2, w_time, Wt1, bt1, Wt2,
           bt2, interpret=False):
    full = lambda shape: pl.BlockSpec(shape, lambda b, i: (0,) * len(shape))
    return pl.pallas_call(
        _fused_kernel,
        grid=(B, NBLK),
        in_specs=[
            full((B, 1)),
            full((B, 1, S)),
            full((B * 64, 32)),
            full((64, E)),
            full((1, E)),
            full((32, 32)), full((1, 32)),
            full((32, E)), full((1, E)),
            full((1, 128)),
            full((256, E)), full((1, E)),
            pl.BlockSpec(memory_space=pl.ANY),
            full((1, E)),
            pl.BlockSpec((None, RB, E), lambda b, i: (b, i, 0)),
        ],
        out_specs=pl.BlockSpec((None, RB, E), lambda b, i: (b, i, 0)),
        out_shape=jax.ShapeDtypeStruct((B, ROWS_OUT, E), jnp.float32),
        scratch_shapes=[
            pltpu.SMEM((B,), jnp.int32),
            pltpu.VMEM((B, T, E), jnp.float32),
            pltpu.VMEM((E, E), jnp.float32),
            pltpu.SemaphoreType.DMA,
        ],
        interpret=interpret,
    )(timesteps.astype(jnp.float32).reshape(B, 1),
      attention_mask.reshape(B, 1, S),
      noisy_actions.reshape(B * 64, 32),
      noise_pos.reshape(64, E), timestep_pos.reshape(1, E),
      W1, b1.reshape(1, 32), W2, b2.reshape(1, E),
      w_time.reshape(1, 128), Wt1, bt1.reshape(1, E), Wt2, bt2.reshape(1, E),
      inputs_embeds)


def kernel(noisy_actions, timesteps, input_ids, attention_mask, inputs_embeds,
           noise_pos, timestep_pos, W1, b1, W2, b2, w_time, Wt1, bt1, Wt2,
           bt2):
    return _fused(noisy_actions, timesteps, attention_mask, inputs_embeds,
                  noise_pos, timestep_pos, W1, b1, W2, b2, w_time, Wt1, bt1,
                  Wt2, bt2)


# speedup vs baseline: 1.0013x; 1.0013x over previous
import jax
import jax.numpy as jnp
from jax.experimental import pallas as pl
from jax.experimental.pallas import tpu as pltpu

B = 8
S = 2048
T = 65
E = 2048
ROWS_OUT = S + T
RB = 896
NBLK = (ROWS_OUT + RB - 1) // RB
WIN = 72


def _fused_kernel(ts_ref, mask_ref, na_ref, npos_ref, tpos_ref,
                  w1_ref, b1_ref, w2_ref, b2_ref, wt_ref,
                  wt1_ref, bt1_ref, wt2_hbm, bt2_ref,
                  in_ref, out_ref, vl_sm, dit_s, wt2_s, sem):
    b = pl.program_id(0)
    i = pl.program_id(1)

    @pl.when((b == 0) & (i == 0))
    def _():
        cp = pltpu.make_async_copy(wt2_hbm, wt2_s, sem)
        cp.start()
        for bb in range(B):
            vl_sm[bb] = jnp.sum(mask_ref[bb])
        g = na_ref[...] @ w1_ref[...] + b1_ref[...]
        g = jax.nn.gelu(g, approximate=True)
        act = g @ w2_ref[...] + b2_ref[...]
        x = ts_ref[...] * wt_ref[...]
        x = jnp.concatenate([jnp.cos(x), jnp.sin(x)], axis=1)
        h1 = x @ wt1_ref[...] + bt1_ref[...]
        h1 = h1 * jax.nn.sigmoid(h1)
        cp.wait()
        tt = h1 @ wt2_s[...] + bt2_ref[...] + tpos_ref[...]
        for bb in range(B):
            dit_s[bb] = jnp.concatenate(
                [tt[bb:bb + 1], act[64 * bb:64 * (bb + 1)] + npos_ref[...]],
                axis=0)

    vl = vl_sm[b]
    r0 = i * RB

    @pl.when(r0 + RB <= S)
    def _():
        out_ref[...] = in_ref[...]

    @pl.when(r0 + RB > S)
    def _():
        rows = r0 + jax.lax.broadcasted_iota(jnp.int32, (RB, 1), 0)
        out_ref[...] = jnp.where(rows < S, in_ref[...], 0.0)

    intersects = (vl < r0 + RB) & (vl + T > r0)

    @pl.when(intersects)
    def _():
        lo = jax.lax.max(vl, r0)
        w0 = jax.lax.min(((lo - r0) // 8) * 8, RB - WIN)
        rows = r0 + w0 + jax.lax.broadcasted_iota(jnp.int32, (WIN, 1), 0)
        rel = rows - vl
        in_window = (rel >= 0) & (rel < T)
        keep = jnp.logical_not(in_window) & (rows < S)
        sub = jnp.where(keep, in_ref[pl.ds(w0, WIN), :], 0.0)
        j = jax.lax.broadcasted_iota(jnp.int32, (WIN, T), 1)
        p = ((rel == j) & in_window).astype(jnp.float32)
        win = jax.lax.dot(p, dit_s[b],
                          preferred_element_type=jnp.float32)
        out_ref[pl.ds(w0, WIN), :] = sub + win


def _fused(noisy_actions, timesteps, attention_mask, inputs_embeds,
           noise_pos, timestep_pos, W1, b1, W2, b2, w_time, Wt1, bt1, Wt2,
           bt2, interpret=False):
    full = lambda shape: pl.BlockSpec(shape, lambda b, i: (0,) * len(shape))
    return pl.pallas_call(
        _fused_kernel,
        grid=(B, NBLK),
        in_specs=[
            full((B, 1)),
            full((B, 1, S)),
            full((B * 64, 32)),
            full((64, E)),
            full((1, E)),
            full((32, 32)), full((1, 32)),
            full((32, E)), full((1, E)),
            full((1, 128)),
            full((256, E)), full((1, E)),
            pl.BlockSpec(memory_space=pl.ANY),
            full((1, E)),
            pl.BlockSpec((None, RB, E), lambda b, i: (b, i, 0)),
        ],
        out_specs=pl.BlockSpec((None, RB, E), lambda b, i: (b, i, 0)),
        out_shape=jax.ShapeDtypeStruct((B, ROWS_OUT, E), jnp.float32),
        scratch_shapes=[
            pltpu.SMEM((B,), jnp.int32),
            pltpu.VMEM((B, T, E), jnp.float32),
            pltpu.VMEM((E, E), jnp.float32),
            pltpu.SemaphoreType.DMA,
        ],
        interpret=interpret,
    )(timesteps.astype(jnp.float32).reshape(B, 1),
      attention_mask.reshape(B, 1, S),
      noisy_actions.reshape(B * 64, 32),
      noise_pos.reshape(64, E), timestep_pos.reshape(1, E),
      W1, b1.reshape(1, 32), W2, b2.reshape(1, E),
      w_time.reshape(1, 128), Wt1, bt1.reshape(1, E), Wt2, bt2.reshape(1, E),
      inputs_embeds)


def kernel(noisy_actions, timesteps, input_ids, attention_mask, inputs_embeds,
           noise_pos, timestep_pos, W1, b1, W2, b2, w_time, Wt1, bt1, Wt2,
           bt2):
    return _fused(noisy_actions, timesteps, attention_mask, inputs_embeds,
                  noise_pos, timestep_pos, W1, b1, W2, b2, w_time, Wt1, bt1,
                  Wt2, bt2)
